# Initial kernel scaffold; baseline (speedup 1.0000x reference)
import jax
import jax.numpy as jnp
from jax.experimental import pallas as pl


def kernel(x, edge_index, W1, b1, W2, b2):
    raise NotImplementedError("write your pallas kernel here")


# speedup vs baseline: 5.9632x; 5.9632x over previous
import functools

import jax
import jax.numpy as jnp
from jax import lax
from jax.experimental import pallas as pl
from jax.experimental.pallas import tpu as pltpu
from jax.experimental.pallas import tpu_sc as plsc

N_NODES = 10000
D = 256
DH = 128

NC = 2
NT = 16
K = 128
NCH = 80
EPT = NCH * K
E_PAD = NT * EPT
ACC_ROWS = 10240
TRASH = N_NODES
ZROWS = ACC_ROWS // NT
WB = ACC_ROWS // NT

BM = 1000
GRID = N_NODES // BM

_MESH = plsc.VectorSubcoreMesh(core_axis_name="c", subcore_axis_name="s")


@functools.partial(
    pl.kernel,
    mesh=_MESH,
    out_type=jax.ShapeDtypeStruct((ACC_ROWS, DH), jnp.float32),
    scratch_types=[
        pltpu.VMEM((NCH, K), jnp.int32),
        pltpu.VMEM((K, DH), jnp.float32),
        pltpu.VMEM_SHARED((ACC_ROWS, DH), jnp.float32),
    ],
)
def _deg_kernel(col_hbm, ones_hbm, zeros_hbm, out_hbm, colv, onesv, hist):
    c = lax.axis_index("c")
    s = lax.axis_index("s")

    @pl.when(c == 0)
    def _():
        pltpu.sync_copy(col_hbm.at[s], colv)
        pltpu.sync_copy(ones_hbm, onesv)
        pltpu.sync_copy(zeros_hbm, hist.at[pl.ds(s * ZROWS, ZROWS)])
        plsc.subcore_barrier()

        def body(j, carry):
            pltpu.sync_copy(onesv, hist.at[colv.at[j]], add=True)
            return carry

        lax.fori_loop(0, NCH, body, 0)
        plsc.subcore_barrier()
        pltpu.sync_copy(hist.at[pl.ds(s * ZROWS, ZROWS)],
                        out_hbm.at[pl.ds(s * ZROWS, ZROWS)])


@functools.partial(
    pl.kernel,
    mesh=_MESH,
    out_type=jax.ShapeDtypeStruct((NC, ACC_ROWS, DH), jnp.float32),
    scratch_types=[
        pltpu.VMEM((NCH, K), jnp.int32),
        pltpu.VMEM((NCH, K), jnp.int32),
        pltpu.VMEM((K, DH), jnp.float32),
        pltpu.SemaphoreType.DMA,
        pltpu.VMEM_SHARED((ACC_ROWS, DH), jnp.float32),
    ],
)
def _agg_kernel(g_hbm, row2_hbm, col_hbm, zeros_hbm, out_hbm,
                rowv, colv, buf, sem, acc):
    c = lax.axis_index("c")
    s = lax.axis_index("s")
    pltpu.sync_copy(row2_hbm.at[c].at[s], rowv)
    pltpu.sync_copy(col_hbm.at[s], colv)
    pltpu.sync_copy(zeros_hbm, acc.at[pl.ds(s * ZROWS, ZROWS)])
    plsc.subcore_barrier()

    def body(j, carry):
        pltpu.async_copy(g_hbm.at[rowv.at[j]], buf, sem).wait()
        pltpu.sync_copy(buf, acc.at[colv.at[j]], add=True)
        return carry

    lax.fori_loop(0, NCH, body, 0)
    plsc.subcore_barrier()
    pltpu.sync_copy(acc.at[pl.ds(s * WB, WB)],
                    out_hbm.at[c].at[pl.ds(s * WB, WB)])


def _rows(i):
    return (i, 0)


def _mm_body(x_ref, w_ref, deg_ref, g_ref):
    dinv = lax.rsqrt(deg_ref[:, 0:1] + 1.0)
    g_ref[...] = jnp.dot(x_ref[...], w_ref[...],
                         preferred_element_type=jnp.float32) * dinv


def _mm_call(x, W, deg16):
    return pl.pallas_call(
        _mm_body,
        grid=(GRID,),
        in_specs=[
            pl.BlockSpec((BM, D), _rows),
            pl.BlockSpec((D, D), lambda i: (0, 0)),
            pl.BlockSpec((BM, 16), _rows),
        ],
        out_specs=pl.BlockSpec((BM, D), _rows),
        out_shape=jax.ShapeDtypeStruct((N_NODES, D), jnp.float32),
    )(x, W, deg16)


def _fused_body(s0_ref, s1_ref, g_ref, deg_ref, b_ref, w_ref, h_ref, g2_ref):
    dinv = lax.rsqrt(deg_ref[:, 0:1] + 1.0)
    S = jnp.concatenate([s0_ref[...], s1_ref[...]], axis=1)
    h = jnp.maximum(dinv * (S + g_ref[...]) + b_ref[...], 0.0)
    h_ref[...] = h
    g2_ref[...] = jnp.dot(h, w_ref[...],
                          preferred_element_type=jnp.float32) * dinv


def _fused_call(s0, s1, g, deg16, b, W2):
    return pl.pallas_call(
        _fused_body,
        grid=(GRID,),
        in_specs=[
            pl.BlockSpec((BM, DH), _rows),
            pl.BlockSpec((BM, DH), _rows),
            pl.BlockSpec((BM, D), _rows),
            pl.BlockSpec((BM, 16), _rows),
            pl.BlockSpec((1, D), lambda i: (0, 0)),
            pl.BlockSpec((D, D), lambda i: (0, 0)),
        ],
        out_specs=[
            pl.BlockSpec((BM, D), _rows),
            pl.BlockSpec((BM, D), _rows),
        ],
        out_shape=[
            jax.ShapeDtypeStruct((N_NODES, D), jnp.float32),
            jax.ShapeDtypeStruct((N_NODES, D), jnp.float32),
        ],
    )(s0, s1, g, deg16, b, W2)


def _epi_body(s0_ref, s1_ref, g_ref, deg_ref, b_ref, h_ref):
    dinv = lax.rsqrt(deg_ref[:, 0:1] + 1.0)
    S = jnp.concatenate([s0_ref[...], s1_ref[...]], axis=1)
    h_ref[...] = jnp.maximum(dinv * (S + g_ref[...]) + b_ref[...], 0.0)


def _epi_call(s0, s1, g, deg16, b):
    return pl.pallas_call(
        _epi_body,
        grid=(GRID,),
        in_specs=[
            pl.BlockSpec((BM, DH), _rows),
            pl.BlockSpec((BM, DH), _rows),
            pl.BlockSpec((BM, D), _rows),
            pl.BlockSpec((BM, 16), _rows),
            pl.BlockSpec((1, D), lambda i: (0, 0)),
        ],
        out_specs=pl.BlockSpec((BM, D), _rows),
        out_shape=jax.ShapeDtypeStruct((N_NODES, D), jnp.float32),
    )(s0, s1, g, deg16, b)


def kernel(x, edge_index, W1, b1, W2, b2):
    x = x.astype(jnp.float32)
    row = edge_index[0].astype(jnp.int32)
    col = edge_index[1].astype(jnp.int32)
    pad = E_PAD - row.shape[0]
    rowp = jnp.concatenate([row, jnp.zeros((pad,), jnp.int32)])
    colp = jnp.concatenate([col, jnp.full((pad,), TRASH, jnp.int32)])
    row2 = jnp.stack([rowp * 2, rowp * 2 + 1]).reshape(NC, NT, NCH, K)
    col3 = colp.reshape(NT, NCH, K)

    ones_k = jnp.ones((K, DH), jnp.float32)
    zeros_dh = jnp.zeros((ZROWS, DH), jnp.float32)

    deg16 = _deg_kernel(col3, ones_k, zeros_dh)[:N_NODES, :16]

    g1 = _mm_call(x, W1, deg16)
    S1 = _agg_kernel(g1.reshape(2 * N_NODES, DH), row2, col3, zeros_dh)
    h1, g2 = _fused_call(S1[0, :N_NODES], S1[1, :N_NODES], g1, deg16,
                         b1.reshape(1, D), W2)
    S2 = _agg_kernel(g2.reshape(2 * N_NODES, DH), row2, col3, zeros_dh)
    h2 = _epi_call(S2[0, :N_NODES], S2[1, :N_NODES], g2, deg16,
                   b2.reshape(1, D))
    return jnp.concatenate([h1, h2], axis=1)
